# Initial kernel scaffold; baseline (speedup 1.0000x reference)
import jax
import jax.numpy as jnp
from jax.experimental import pallas as pl


def kernel(x, table):
    raise NotImplementedError("write your pallas kernel here")


# speedup vs baseline: 2.7679x; 2.7679x over previous
import functools

import jax
import jax.numpy as jnp
from jax import lax
from jax.experimental import pallas as pl
from jax.experimental.pallas import tpu as pltpu
from jax.experimental.pallas import tpu_sc as plsc

VOCAB = 100000
EMB = 128
N_TOKENS = 4096 * 50

_NC = 2
_NS = 16
_NW = _NC * _NS
_PER_W = N_TOKENS // _NW
_CHUNK = 128
_NCHUNK = _PER_W // _CHUNK


def _emb_body(x_hbm, table_hbm, out_hbm, idx_v, rows_v, sem):
    wid = lax.axis_index("s") * _NC + lax.axis_index("c")
    base = wid * _PER_W

    def chunk(i, carry):
        off = base + i * _CHUNK
        pltpu.sync_copy(x_hbm.at[pl.ds(off, _CHUNK)], idx_v)
        pltpu.async_copy(table_hbm.at[idx_v], rows_v, sem).wait()
        pltpu.sync_copy(rows_v, out_hbm.at[pl.ds(off, _CHUNK)])
        return carry

    lax.fori_loop(0, _NCHUNK, chunk, 0)


_mesh = plsc.VectorSubcoreMesh(core_axis_name="c", subcore_axis_name="s")

_emb_kernel = functools.partial(
    pl.kernel,
    mesh=_mesh,
    out_type=jax.ShapeDtypeStruct((N_TOKENS, EMB), jnp.float32),
    scratch_types=[
        pltpu.VMEM((_CHUNK,), jnp.int32),
        pltpu.VMEM((_CHUNK, EMB), jnp.float32),
        pltpu.SemaphoreType.DMA,
    ],
)(_emb_body)


def kernel(x, table):
    flat = x.reshape(-1).astype(jnp.int32)
    out = _emb_kernel(flat, table)
    return out.reshape(x.shape[0], x.shape[1], EMB)


# speedup vs baseline: 3.3177x; 1.1986x over previous
import functools

import jax
import jax.numpy as jnp
from jax import lax
from jax.experimental import pallas as pl
from jax.experimental.pallas import tpu as pltpu
from jax.experimental.pallas import tpu_sc as plsc

VOCAB = 100000
EMB = 128
N_TOKENS = 4096 * 50

_NC = 2
_NS = 16
_NW = _NC * _NS
_PER_W = N_TOKENS // _NW
_CHUNK = 128
_NCHUNK = _PER_W // _CHUNK
_NBUF = 5
_NGROUP = _NCHUNK // _NBUF


def _emb_body(x_hbm, table_hbm, out_hbm, idx_v, rows_v, gsem, osem):
    wid = lax.axis_index("s") * _NC + lax.axis_index("c")
    base = wid * _PER_W
    pltpu.sync_copy(x_hbm.at[wid], idx_v)

    def start_gather(j, b):
        pltpu.async_copy(table_hbm.at[idx_v.at[j]], rows_v.at[b], gsem.at[b])

    def wait_gather(b):
        pltpu.make_async_copy(
            table_hbm.at[pl.ds(0, _CHUNK)], rows_v.at[b], gsem.at[b]
        ).wait()

    def start_out(j, b):
        pltpu.async_copy(
            rows_v.at[b], out_hbm.at[pl.ds(base + j * _CHUNK, _CHUNK)], osem.at[b]
        )

    def wait_out(b):
        pltpu.make_async_copy(
            rows_v.at[b], out_hbm.at[pl.ds(0, _CHUNK)], osem.at[b]
        ).wait()

    for b in range(_NBUF):
        start_gather(b, b)
    for b in range(_NBUF):
        wait_gather(b)
        start_out(b, b)

    def group(g, carry):
        j0 = g * _NBUF
        for b in range(_NBUF):
            wait_out(b)
            start_gather(j0 + b, b)
        for b in range(_NBUF):
            wait_gather(b)
            start_out(j0 + b, b)
        return carry

    lax.fori_loop(1, _NGROUP, group, 0)

    for b in range(_NBUF):
        wait_out(b)


_mesh = plsc.VectorSubcoreMesh(core_axis_name="c", subcore_axis_name="s")

_emb_kernel = functools.partial(
    pl.kernel,
    mesh=_mesh,
    out_type=jax.ShapeDtypeStruct((N_TOKENS, EMB), jnp.float32),
    scratch_types=[
        pltpu.VMEM((_NCHUNK, _CHUNK), jnp.int32),
        pltpu.VMEM((_NBUF, _CHUNK, EMB), jnp.float32),
        pltpu.SemaphoreType.DMA((_NBUF,)),
        pltpu.SemaphoreType.DMA((_NBUF,)),
    ],
)(_emb_body)


def kernel(x, table):
    flat = x.reshape(-1).astype(jnp.int32).reshape(_NW, _NCHUNK, _CHUNK)
    out = _emb_kernel(flat, table)
    return out.reshape(x.shape[0], x.shape[1], EMB)
